# baseline (speedup 1.0000x reference)
import functools

import jax
import jax.numpy as jnp
from jax import lax
from jax.experimental import pallas as pl
from jax.experimental.pallas import tpu as pltpu
from jax.experimental.pallas import tpu_sc as plsc

_DIM = 256
_NNZ = _DIM * (_DIM + 1) // 2
_BATCH = 1024
_NTILES = 32
_PER_W = _BATCH // _NTILES
_HALF = _DIM // 2
_CNT_A = sum(_DIM - i for i in range(_HALF))
_CNT_B = _NNZ - _CNT_A
_PAD = 128


def _tri_body(
    x_hbm, out_hbm,
    buf_a, buf_b, out_v,
    sem_a, sem_b, sem_out, sem_out2,
):
    nc = 2
    wid = lax.axis_index("s") * nc + lax.axis_index("c")
    b0 = wid * _PER_W

    def stage_a(b):
        return pltpu.make_async_copy(
            x_hbm.at[b, pl.ds(0, _HALF)], buf_a, sem_a
        )

    def stage_b(b):
        return pltpu.make_async_copy(
            x_hbm.at[b, pl.ds(_HALF, _HALF), pl.ds(_HALF, _HALF)],
            buf_b,
            sem_b,
        )

    def out_copy_a(b):
        return pltpu.make_async_copy(
            out_v.at[pl.ds(0, _CNT_A)],
            out_hbm.at[pl.ds(b * _NNZ, _CNT_A)],
            sem_out,
        )

    def out_copy_b(b):
        return pltpu.make_async_copy(
            out_v.at[pl.ds(_CNT_A, _CNT_B)],
            out_hbm.at[pl.ds(b * _NNZ + _CNT_A, _CNT_B)],
            sem_out2,
        )

    iota = lax.iota(jnp.int32, 16)

    def gather_phase(buf, width, off0):
        nchunks = width // 16

        def row_body(li, off):
            rvec = jnp.broadcast_to(li, (16,))
            colbase = iota + li
            for c in range(nchunks):
                col = colbase + (16 * c)
                out_v[pl.ds(off + 16 * c, 16)] = plsc.load_gather(
                    buf, [rvec, col]
                )
            return off + (width - li)

        lax.fori_loop(0, _HALF, row_body, off0)

    stage_a(b0).start()
    stage_b(b0).start()

    def batch_body(bl, carry):
        b = b0 + bl
        stage_a(b).wait()

        @pl.when(bl > 0)
        def _():
            out_copy_a(b - 1).wait()
            out_copy_b(b - 1).wait()

        gather_phase(buf_a, _DIM, 0)

        @pl.when(bl < _PER_W - 1)
        def _():
            stage_a(b + 1).start()

        out_copy_a(b).start()
        stage_b(b).wait()
        gather_phase(buf_b, _HALF, _CNT_A)

        @pl.when(bl < _PER_W - 1)
        def _():
            stage_b(b + 1).start()

        out_copy_b(b).start()
        return carry

    lax.fori_loop(0, _PER_W, batch_body, 0)
    out_copy_a(b0 + _PER_W - 1).wait()
    out_copy_b(b0 + _PER_W - 1).wait()


@jax.jit
def _tritovec(x):
    mesh = plsc.VectorSubcoreMesh(core_axis_name="c", subcore_axis_name="s")
    fn = functools.partial(
        pl.kernel,
        mesh=mesh,
        out_type=jax.ShapeDtypeStruct((_BATCH * _NNZ,), jnp.float32),
        scratch_types=[
            pltpu.VMEM((_HALF, _DIM), jnp.float32),
            pltpu.VMEM((_HALF, _HALF), jnp.float32),
            pltpu.VMEM((_NNZ + _PAD,), jnp.float32),
            pltpu.SemaphoreType.DMA,
            pltpu.SemaphoreType.DMA,
            pltpu.SemaphoreType.DMA,
            pltpu.SemaphoreType.DMA,
        ],
        compiler_params=pltpu.CompilerParams(
            use_tc_tiling_on_sc=True, needs_layout_passes=False
        ),
    )(_tri_body)
    return fn(x)


def kernel(input):
    return _tritovec(input).reshape(_BATCH, _NNZ, 1)


# speedup vs baseline: 2.2189x; 2.2189x over previous
import functools

import jax
import jax.numpy as jnp
import numpy as np
from jax import lax
from jax.experimental import pallas as pl
from jax.experimental.pallas import tpu as pltpu
from jax.experimental.pallas import tpu_sc as plsc

_DIM = 256
_NNZ = _DIM * (_DIM + 1) // 2
_BATCH = 1024
_NTILES = 32
_PER_W = _BATCH // _NTILES
_HALF = _DIM // 2
_CNT_A = sum(_DIM - i for i in range(_HALF))
_CNT_B = _NNZ - _CNT_A


def _packed_triu_idx() -> np.ndarray:
    i, j = np.triu_indices(_DIM)
    a = i < _HALF
    idx_a = (i[a] << 8) | j[a]
    idx_b = ((i[~a] - _HALF) << 8) | (j[~a] - _HALF)
    idx = np.concatenate([idx_a, idx_b]).astype(np.int32)
    pairs = idx.reshape(-1, 2, 16)
    return (pairs[:, 0, :] | (pairs[:, 1, :] << 16)).reshape(-1)


_IDX = _packed_triu_idx()


def _tri_body(
    x_hbm, idx_hbm, out_hbm,
    idx_v, buf_a, buf_b, out_v,
    sem_a, sem_b, sem_out, sem_out2,
):
    nc = 2
    wid = lax.axis_index("s") * nc + lax.axis_index("c")
    b0 = wid * _PER_W

    def stage_a(b):
        return pltpu.make_async_copy(
            x_hbm.at[b, pl.ds(0, _HALF)], buf_a, sem_a
        )

    def stage_b(b):
        return pltpu.make_async_copy(
            x_hbm.at[b, pl.ds(_HALF, _HALF), pl.ds(_HALF, _HALF)],
            buf_b,
            sem_b,
        )

    def out_copy_a(b):
        return pltpu.make_async_copy(
            out_v.at[pl.ds(0, _CNT_A)],
            out_hbm.at[pl.ds(b * _NNZ, _CNT_A)],
            sem_out,
        )

    def out_copy_b(b):
        return pltpu.make_async_copy(
            out_v.at[pl.ds(_CNT_A, _CNT_B)],
            out_hbm.at[pl.ds(b * _NNZ + _CNT_A, _CNT_B)],
            sem_out2,
        )

    def gather_span(buf, pair0, npairs):
        @plsc.parallel_loop(0, npairs, unroll=8)
        def _(ci):
            base = (pair0 + ci) * 16
            iv = idx_v[pl.ds(base, 16)]
            lo = lax.bitwise_and(iv, 0xFFFF)
            hi = lax.shift_right_logical(iv, 16)
            r0 = lax.shift_right_logical(lo, 8)
            c0 = lax.bitwise_and(lo, 255)
            r1 = lax.shift_right_logical(hi, 8)
            c1 = lax.bitwise_and(hi, 255)
            out_v[pl.ds(base * 2, 16)] = plsc.load_gather(buf, [r0, c0])
            out_v[pl.ds(base * 2 + 16, 16)] = plsc.load_gather(buf, [r1, c1])

    stage_a(b0).start()
    stage_b(b0).start()
    pltpu.sync_copy(idx_hbm, idx_v)

    def batch_body(bl, carry):
        b = b0 + bl
        stage_a(b).wait()

        @pl.when(bl > 0)
        def _():
            out_copy_a(b - 1).wait()

        gather_span(buf_a, 0, _CNT_A // 32)

        @pl.when(bl < _PER_W - 1)
        def _():
            stage_a(b + 1).start()

        out_copy_a(b).start()
        stage_b(b).wait()

        @pl.when(bl > 0)
        def _():
            out_copy_b(b - 1).wait()

        gather_span(buf_b, _CNT_A // 32, _CNT_B // 32)

        @pl.when(bl < _PER_W - 1)
        def _():
            stage_b(b + 1).start()

        out_copy_b(b).start()
        return carry

    lax.fori_loop(0, _PER_W, batch_body, 0)
    out_copy_a(b0 + _PER_W - 1).wait()
    out_copy_b(b0 + _PER_W - 1).wait()


@jax.jit
def _tritovec(x, idx):
    mesh = plsc.VectorSubcoreMesh(core_axis_name="c", subcore_axis_name="s")
    fn = functools.partial(
        pl.kernel,
        mesh=mesh,
        out_type=jax.ShapeDtypeStruct((_BATCH * _NNZ,), jnp.float32),
        scratch_types=[
            pltpu.VMEM((_NNZ // 2,), jnp.int32),
            pltpu.VMEM((_HALF, _DIM), jnp.float32),
            pltpu.VMEM((_HALF, _HALF), jnp.float32),
            pltpu.VMEM((_NNZ,), jnp.float32),
            pltpu.SemaphoreType.DMA,
            pltpu.SemaphoreType.DMA,
            pltpu.SemaphoreType.DMA,
            pltpu.SemaphoreType.DMA,
        ],
        compiler_params=pltpu.CompilerParams(
            use_tc_tiling_on_sc=True, needs_layout_passes=False
        ),
    )(_tri_body)
    return fn(x, idx)


def kernel(input):
    idx = jnp.asarray(_IDX)
    return _tritovec(input, idx).reshape(_BATCH, _NNZ, 1)


# speedup vs baseline: 2.3546x; 1.0612x over previous
import functools

import jax
import jax.numpy as jnp
import numpy as np
from jax import lax
from jax.experimental import pallas as pl
from jax.experimental.pallas import tpu as pltpu
from jax.experimental.pallas import tpu_sc as plsc

_DIM = 256
_NNZ = _DIM * (_DIM + 1) // 2
_BATCH = 1024
_NTILES = 32
_PER_W = _BATCH // _NTILES
_HALF = _DIM // 2
_CNT_A = sum(_DIM - i for i in range(_HALF))
_CNT_B = _NNZ - _CNT_A


def _packed_triu_idx() -> np.ndarray:
    i, j = np.triu_indices(_DIM)
    a = i < _HALF
    idx_a = (i[a] << 8) | j[a]
    idx_b = ((i[~a] - _HALF) << 8) | (j[~a] - _HALF)
    return np.concatenate([idx_a, idx_b]).astype(np.int32)


_IDX = _packed_triu_idx()


def _tri_body(
    x_hbm, idx_hbm, out_hbm,
    idx_v, buf_a, buf_b, out_v,
    sem_a, sem_b, sem_out, sem_out2,
):
    nc = 2
    wid = lax.axis_index("s") * nc + lax.axis_index("c")
    b0 = wid * _PER_W

    def stage_a(b):
        return pltpu.make_async_copy(
            x_hbm.at[b, pl.ds(0, _HALF)], buf_a, sem_a
        )

    def stage_b(b):
        return pltpu.make_async_copy(
            x_hbm.at[b, pl.ds(_HALF, _HALF), pl.ds(_HALF, _HALF)],
            buf_b,
            sem_b,
        )

    def out_copy_a(b):
        return pltpu.make_async_copy(
            out_v.at[pl.ds(0, _CNT_A)],
            out_hbm.at[pl.ds(b * _NNZ, _CNT_A)],
            sem_out,
        )

    def out_copy_b(b):
        return pltpu.make_async_copy(
            out_v.at[pl.ds(_CNT_A, _CNT_B)],
            out_hbm.at[pl.ds(b * _NNZ + _CNT_A, _CNT_B)],
            sem_out2,
        )

    def gather_span(buf, chunk0, nchunks):
        group = 4

        @plsc.parallel_loop(0, nchunks // group, unroll=4)
        def _(cg):
            base = (chunk0 + cg * group) * 16
            ivs = [idx_v[pl.ds(base + 16 * u, 16)] for u in range(group)]
            rs = [lax.shift_right_logical(iv, 8) for iv in ivs]
            cs = [lax.bitwise_and(iv, 255) for iv in ivs]
            gs = [
                plsc.load_gather(buf, [rs[u], cs[u]]) for u in range(group)
            ]
            for u in range(group):
                out_v[pl.ds(base + 16 * u, 16)] = gs[u]

    stage_a(b0).start()
    stage_b(b0).start()
    pltpu.sync_copy(idx_hbm, idx_v)

    def batch_body(bl, carry):
        b = b0 + bl
        stage_a(b).wait()

        @pl.when(bl > 0)
        def _():
            out_copy_a(b - 1).wait()

        gather_span(buf_a, 0, _CNT_A // 16)

        @pl.when(bl < _PER_W - 1)
        def _():
            stage_a(b + 1).start()

        out_copy_a(b).start()
        stage_b(b).wait()

        @pl.when(bl > 0)
        def _():
            out_copy_b(b - 1).wait()

        gather_span(buf_b, _CNT_A // 16, _CNT_B // 16)

        @pl.when(bl < _PER_W - 1)
        def _():
            stage_b(b + 1).start()

        out_copy_b(b).start()
        return carry

    lax.fori_loop(0, _PER_W, batch_body, 0)
    out_copy_a(b0 + _PER_W - 1).wait()
    out_copy_b(b0 + _PER_W - 1).wait()


@jax.jit
def _tritovec(x, idx):
    mesh = plsc.VectorSubcoreMesh(core_axis_name="c", subcore_axis_name="s")
    fn = functools.partial(
        pl.kernel,
        mesh=mesh,
        out_type=jax.ShapeDtypeStruct((_BATCH * _NNZ,), jnp.float32),
        scratch_types=[
            pltpu.VMEM((_NNZ,), jnp.int32),
            pltpu.VMEM((_HALF, _DIM), jnp.float32),
            pltpu.VMEM((_HALF, _HALF), jnp.float32),
            pltpu.VMEM((_NNZ,), jnp.float32),
            pltpu.SemaphoreType.DMA,
            pltpu.SemaphoreType.DMA,
            pltpu.SemaphoreType.DMA,
            pltpu.SemaphoreType.DMA,
        ],
        compiler_params=pltpu.CompilerParams(
            use_tc_tiling_on_sc=True, needs_layout_passes=False
        ),
    )(_tri_body)
    return fn(x, idx)


def kernel(input):
    idx = jnp.asarray(_IDX)
    return _tritovec(input, idx).reshape(_BATCH, _NNZ, 1)
